# baseline (speedup 1.0000x reference)
import functools
import math

import jax
import jax.numpy as jnp
from jax.experimental import pallas as pl
from jax.experimental.pallas import tpu as pltpu

N_EXP = 8
TOP_K = 1
CAPACITY_FACTOR = 1.0
MIN_CAPACITY = 4

BT = 256


def _capacity(num_tokens: int) -> int:
    capacity = math.floor(TOP_K * CAPACITY_FACTOR * num_tokens / N_EXP)
    capacity += capacity % 2
    return int(max(capacity, MIN_CAPACITY))


def _router_body(capacity, x_ref, wg_ref, uc_ref, cb_ref, secw_ref, counts_ref):
    i = pl.program_id(0)

    @pl.when(i == 0)
    def _init():
        counts_ref[...] = jnp.zeros_like(counts_ref)

    logits = jax.lax.dot_general(
        x_ref[...], wg_ref[...],
        dimension_numbers=(((1,), (1,)), ((), ())),
        preferred_element_type=jnp.float32,
    )

    m = jnp.max(logits, axis=1, keepdims=True)
    eidx = jax.lax.broadcasted_iota(jnp.int32, (BT, N_EXP), 1)
    e = jnp.min(jnp.where(logits >= m, eidx, N_EXP), axis=1, keepdims=True)
    oh = (eidx == e).astype(jnp.float32)

    ri = jax.lax.broadcasted_iota(jnp.int32, (BT, BT), 0)
    ci = jax.lax.broadcasted_iota(jnp.int32, (BT, BT), 1)
    ltri = (ci < ri).astype(jnp.float32)
    prior = jax.lax.dot_general(
        ltri, oh, dimension_numbers=(((1,), (0,)), ((), ())),
        preferred_element_type=jnp.float32,
    )

    base = counts_ref[...]
    slot_all = prior.astype(jnp.int32) + base
    slots = jnp.sum(jnp.where(eidx == e, slot_all, 0), axis=1, keepdims=True)

    new_counts = base + jnp.sum(oh, axis=0, keepdims=True).astype(jnp.int32)
    counts_ref[...] = new_counts
    uc_ref[...] = jnp.minimum(new_counts, capacity)

    p = jnp.where(slots >= capacity, -1, e * capacity + slots)
    j = jax.lax.broadcasted_iota(jnp.int32, (BT, N_EXP * capacity), 1)
    cb_ref[...] = (j == p).astype(jnp.float32)

    jw = jax.lax.broadcasted_iota(jnp.int32, (BT, N_EXP * capacity // 4), 1)
    byte_val = jnp.left_shift(jnp.int32(1), 8 * jnp.bitwise_and(p, 3))
    secw_ref[...] = jnp.where(jw == jnp.right_shift(p, 2), byte_val, 0)


def kernel(x, w_g):
    num_tokens, n_embd = x.shape
    capacity = _capacity(num_tokens)
    grid = (num_tokens // BT,)
    body = functools.partial(_router_body, capacity)

    uc2, cb2, secw = pl.pallas_call(
        body,
        grid=grid,
        in_specs=[
            pl.BlockSpec((BT, n_embd), lambda i: (i, 0)),
            pl.BlockSpec((N_EXP, n_embd), lambda i: (0, 0)),
        ],
        out_specs=[
            pl.BlockSpec((1, N_EXP), lambda i: (0, 0)),
            pl.BlockSpec((BT, N_EXP * capacity), lambda i: (i, 0)),
            pl.BlockSpec((BT, N_EXP * capacity // 4), lambda i: (i, 0)),
        ],
        out_shape=[
            jax.ShapeDtypeStruct((1, N_EXP), jnp.int32),
            jax.ShapeDtypeStruct((num_tokens, N_EXP * capacity), jnp.float32),
            jax.ShapeDtypeStruct((num_tokens, N_EXP * capacity // 4), jnp.int32),
        ],
        scratch_shapes=[pltpu.VMEM((1, N_EXP), jnp.int32)],
    )(x, w_g)

    used_capacity = uc2.reshape(N_EXP)
    cb_weight = cb2.reshape(num_tokens, N_EXP, capacity)
    sec_bytes = jax.lax.bitcast_convert_type(secw, jnp.uint8)
    sec_mask = sec_bytes.reshape(num_tokens, N_EXP, capacity).astype(jnp.bool_)
    return used_capacity, cb_weight, sec_mask


# speedup vs baseline: 1.5196x; 1.5196x over previous
import functools
import math

import jax
import jax.numpy as jnp
from jax import lax
from jax.experimental import pallas as pl
from jax.experimental.pallas import tpu as pltpu
from jax.experimental.pallas import tpu_sc as plsc

N_EXP = 8
TOP_K = 1
CAPACITY_FACTOR = 1.0
MIN_CAPACITY = 4

BT = 1024
BTS = 256

ROWS_PER_WORKER = 128
CHUNK = 16


def _capacity(num_tokens: int) -> int:
    capacity = math.floor(TOP_K * CAPACITY_FACTOR * num_tokens / N_EXP)
    capacity += capacity % 2
    return int(max(capacity, MIN_CAPACITY))


def _router_body(capacity, x_ref, wg_ref, uc_ref, p_ref, counts_ref):
    i = pl.program_id(0)

    @pl.when(i == 0)
    def _init():
        counts_ref[...] = jnp.zeros_like(counts_ref)

    logits = jax.lax.dot_general(
        wg_ref[...], x_ref[...],
        dimension_numbers=(((1,), (1,)), ((), ())),
        preferred_element_type=jnp.float32,
    )

    m = jnp.max(logits, axis=0, keepdims=True)
    eidx = jax.lax.broadcasted_iota(jnp.int32, (N_EXP, BT), 0)
    e = jnp.min(jnp.where(logits >= m, eidx, N_EXP), axis=0,
                keepdims=True)
    oh = (eidx == e).astype(jnp.float32)

    ri = jax.lax.broadcasted_iota(jnp.int32, (BT, BT), 0)
    ci = jax.lax.broadcasted_iota(jnp.int32, (BT, BT), 1)
    utri = (ri < ci).astype(jnp.float32)
    prior = jax.lax.dot_general(
        oh, utri, dimension_numbers=(((1,), (0,)), ((), ())),
        preferred_element_type=jnp.float32,
    )

    base = counts_ref[:, 0:1]
    slot_all = prior.astype(jnp.int32) + base
    slots = jnp.sum(jnp.where(eidx == e, slot_all, 0), axis=0,
                    keepdims=True)

    new_counts = base + jnp.sum(oh, axis=1, keepdims=True).astype(jnp.int32)
    counts_ref[...] = jnp.broadcast_to(new_counts, counts_ref.shape)
    uc_ref[...] = jnp.broadcast_to(
        jnp.minimum(new_counts, capacity), uc_ref.shape)

    p = jnp.where(slots >= capacity, -1, e * capacity + slots)
    p_ref[0, pl.ds(i * BT, BT)] = p[0, :]


def _cb_body(num_chunks, wide, p_hbm, eye_hbm, zeros_hbm, cb_hbm,
             p_v, payload, stage):
    info = plsc.get_sparse_core_info()
    wid = lax.axis_index("s") * info.num_cores + lax.axis_index("c")
    base = pl.multiple_of(wid * ROWS_PER_WORKER, ROWS_PER_WORKER)
    pltpu.sync_copy(p_hbm.at[0, pl.ds(base, ROWS_PER_WORKER)], p_v)
    pltpu.sync_copy(zeros_hbm, stage)

    lanes = lax.iota(jnp.int32, 16)
    rows_per_chunk = CHUNK * wide // 128

    for c in range(num_chunks):
        off = pl.multiple_of(base * (wide // 128) + c * rows_per_chunk,
                             rows_per_chunk)
        pltpu.sync_copy(stage, cb_hbm.at[pl.ds(off, rows_per_chunk)])

    for c in range(num_chunks):
        cols = p_v[pl.ds(c * CHUNK, CHUNK)]
        valid = cols >= 0
        t_glob = base + c * CHUNK + lanes
        pos = jnp.where(valid, cols, 0)
        w_vec = jax.lax.shift_right_logical(t_glob * wide + pos, 7)
        l_vec = jnp.where(valid, jnp.bitwise_and(cols, 127), 128)
        pltpu.sync_copy(eye_hbm.at[l_vec], payload)
        pltpu.sync_copy(payload, cb_hbm.at[w_vec])


def _sec_body(capacity, p_ref, sec_ref):
    i = pl.program_id(0)
    p_row = p_ref[0, pl.ds(i * BTS, BTS)].reshape(1, BTS).astype(jnp.float32)
    ident = (jax.lax.broadcasted_iota(jnp.int32, (BTS, BTS), 0) ==
             jax.lax.broadcasted_iota(jnp.int32, (BTS, BTS), 1)
             ).astype(jnp.float32)
    p_col = jax.lax.dot_general(
        ident, p_row, dimension_numbers=(((1,), (1,)), ((), ())),
        preferred_element_type=jnp.float32,
    ).astype(jnp.int32)
    j = jax.lax.broadcasted_iota(jnp.int32, (BTS, N_EXP * capacity), 1)
    sec_ref[...] = j == p_col


def kernel(x, w_g):
    num_tokens, n_embd = x.shape
    capacity = _capacity(num_tokens)
    wide = N_EXP * capacity

    uc2, p2 = pl.pallas_call(
        functools.partial(_router_body, capacity),
        grid=(num_tokens // BT,),
        in_specs=[
            pl.BlockSpec((BT, n_embd), lambda i: (i, 0)),
            pl.BlockSpec((N_EXP, n_embd), lambda i: (0, 0)),
        ],
        out_specs=[
            pl.BlockSpec((N_EXP, 128), lambda i: (0, 0)),
            pl.BlockSpec((1, num_tokens), lambda i: (0, 0)),
        ],
        out_shape=[
            jax.ShapeDtypeStruct((N_EXP, 128), jnp.int32),
            jax.ShapeDtypeStruct((1, num_tokens), jnp.int32),
        ],
        scratch_shapes=[pltpu.VMEM((N_EXP, 128), jnp.int32)],
    )(x, w_g)

    num_chunks = ROWS_PER_WORKER // CHUNK
    rows_per_chunk = CHUNK * wide // 128
    eye = jnp.concatenate(
        [jnp.eye(128, dtype=jnp.float32), jnp.zeros((1, 128), jnp.float32)])
    zeros_chunk = jnp.zeros((rows_per_chunk, 128), jnp.float32)
    cb2 = pl.kernel(
        functools.partial(_cb_body, num_chunks, wide),
        out_type=jax.ShapeDtypeStruct((num_tokens * wide // 128, 128),
                                      jnp.float32),
        mesh=plsc.VectorSubcoreMesh(core_axis_name="c", subcore_axis_name="s"),
        scratch_types=[
            pltpu.VMEM((ROWS_PER_WORKER,), jnp.int32),
            pltpu.VMEM((CHUNK, 128), jnp.float32),
            pltpu.VMEM((rows_per_chunk, 128), jnp.float32),
        ],
    )(p2, eye, zeros_chunk)

    sec2 = pl.pallas_call(
        functools.partial(_sec_body, capacity),
        grid=(num_tokens // BTS,),
        in_specs=[pl.BlockSpec((1, num_tokens), lambda i: (0, 0))],
        out_specs=[pl.BlockSpec((BTS, wide), lambda i: (i, 0))],
        out_shape=[jax.ShapeDtypeStruct((num_tokens, wide), jnp.bool_)],
        compiler_params=pltpu.CompilerParams(
            dimension_semantics=("parallel",)),
    )(p2)[0]

    used_capacity = uc2[:, 0]
    cb_weight = cb2.reshape(num_tokens, N_EXP, capacity)
    sec_mask = sec2.reshape(num_tokens, N_EXP, capacity)
    return used_capacity, cb_weight, sec_mask


# speedup vs baseline: 1.7035x; 1.1211x over previous
import functools
import math

import jax
import jax.numpy as jnp
from jax import lax
from jax.experimental import pallas as pl
from jax.experimental.pallas import tpu as pltpu
from jax.experimental.pallas import tpu_sc as plsc

N_EXP = 8
TOP_K = 1
CAPACITY_FACTOR = 1.0
MIN_CAPACITY = 4

BT = 1024
BTS = 256

ROWS_PER_WORKER = 128
CHUNK = 16


def _capacity(num_tokens: int) -> int:
    capacity = math.floor(TOP_K * CAPACITY_FACTOR * num_tokens / N_EXP)
    capacity += capacity % 2
    return int(max(capacity, MIN_CAPACITY))


def _router_body(capacity, x_ref, wg_ref, uc_ref, p_ref, counts_ref):
    i = pl.program_id(0)

    @pl.when(i == 0)
    def _init():
        counts_ref[...] = jnp.zeros_like(counts_ref)

    logits = jax.lax.dot_general(
        wg_ref[...], x_ref[...],
        dimension_numbers=(((1,), (1,)), ((), ())),
        preferred_element_type=jnp.float32,
    )

    m = jnp.max(logits, axis=0, keepdims=True)
    eidx = jax.lax.broadcasted_iota(jnp.int32, (N_EXP, BT), 0)
    e = jnp.min(jnp.where(logits >= m, eidx, N_EXP), axis=0,
                keepdims=True)
    oh = (eidx == e).astype(jnp.float32)

    ri = jax.lax.broadcasted_iota(jnp.int32, (BT, BT), 0)
    ci = jax.lax.broadcasted_iota(jnp.int32, (BT, BT), 1)
    utri = (ri < ci).astype(jnp.float32)
    prior = jax.lax.dot_general(
        oh, utri, dimension_numbers=(((1,), (0,)), ((), ())),
        preferred_element_type=jnp.float32,
    )

    base = counts_ref[:, 0:1]
    slot_all = prior.astype(jnp.int32) + base
    slots = jnp.sum(jnp.where(eidx == e, slot_all, 0), axis=0,
                    keepdims=True)

    new_counts = base + jnp.sum(oh, axis=1, keepdims=True).astype(jnp.int32)
    counts_ref[...] = jnp.broadcast_to(new_counts, counts_ref.shape)
    uc_ref[...] = jnp.broadcast_to(
        jnp.minimum(new_counts, capacity), uc_ref.shape)

    p = jnp.where(slots >= capacity, -1, e * capacity + slots)
    p_ref[0, pl.ds(i * BT, BT)] = p[0, :]


def _sec_sc_body(num_chunks, wide, capacity, p_hbm, eye_hbm, zeros_hbm,
                 sec_hbm, p_v, payload, stage):
    info = plsc.get_sparse_core_info()
    wid = lax.axis_index("s") * info.num_cores + lax.axis_index("c")
    base = pl.multiple_of(wid * ROWS_PER_WORKER, ROWS_PER_WORKER)
    pltpu.sync_copy(p_hbm.at[0, pl.ds(base, ROWS_PER_WORKER)], p_v)
    pltpu.sync_copy(zeros_hbm, stage)

    lanes = lax.iota(jnp.int32, 16)
    rows_per_chunk = CHUNK * N_EXP
    shift = capacity.bit_length() - 1

    for c in range(num_chunks):
        off = pl.multiple_of(base * N_EXP + c * rows_per_chunk,
                             rows_per_chunk)
        pltpu.sync_copy(stage, sec_hbm.at[pl.ds(off, rows_per_chunk)])

    for c in range(num_chunks):
        cols = p_v[pl.ds(c * CHUNK, CHUNK)]
        valid = cols >= 0
        t_glob = base + c * CHUNK + lanes
        pos = jnp.where(valid, cols, 0)
        w_vec = jax.lax.shift_right_logical(t_glob * wide + pos, shift)
        l_vec = jnp.where(valid,
                          jnp.bitwise_and(cols, capacity - 1), capacity)
        pltpu.sync_copy(eye_hbm.at[l_vec], payload)
        pltpu.sync_copy(payload, sec_hbm.at[w_vec])


def _cb_tc_body(capacity, p_ref, cb_ref):
    i = pl.program_id(0)
    p_row = p_ref[0, pl.ds(i * BTS, BTS)].reshape(1, BTS).astype(jnp.float32)
    ident = (jax.lax.broadcasted_iota(jnp.int32, (BTS, BTS), 0) ==
             jax.lax.broadcasted_iota(jnp.int32, (BTS, BTS), 1)
             ).astype(jnp.float32)
    p_col = jax.lax.dot_general(
        ident, p_row, dimension_numbers=(((1,), (1,)), ((), ())),
        preferred_element_type=jnp.float32,
    ).astype(jnp.int32)
    j = jax.lax.broadcasted_iota(jnp.int32, (BTS, N_EXP * capacity), 1)
    cb_ref[...] = (j == p_col).astype(jnp.float32)


def kernel(x, w_g):
    num_tokens, n_embd = x.shape
    capacity = _capacity(num_tokens)
    wide = N_EXP * capacity

    uc2, p2 = pl.pallas_call(
        functools.partial(_router_body, capacity),
        grid=(num_tokens // BT,),
        in_specs=[
            pl.BlockSpec((BT, n_embd), lambda i: (i, 0)),
            pl.BlockSpec((N_EXP, n_embd), lambda i: (0, 0)),
        ],
        out_specs=[
            pl.BlockSpec((N_EXP, 128), lambda i: (0, 0)),
            pl.BlockSpec((1, num_tokens), lambda i: (0, 0)),
        ],
        out_shape=[
            jax.ShapeDtypeStruct((N_EXP, 128), jnp.int32),
            jax.ShapeDtypeStruct((1, num_tokens), jnp.int32),
        ],
        scratch_shapes=[pltpu.VMEM((N_EXP, 128), jnp.int32)],
    )(x, w_g)

    num_chunks = ROWS_PER_WORKER // CHUNK
    rows_per_chunk = CHUNK * N_EXP
    eye = jnp.concatenate(
        [jnp.eye(capacity, dtype=jnp.bool_),
         jnp.zeros((1, capacity), jnp.bool_)])
    zeros_chunk = jnp.zeros((rows_per_chunk, capacity), jnp.bool_)
    sec2 = pl.kernel(
        functools.partial(_sec_sc_body, num_chunks, wide, capacity),
        out_type=jax.ShapeDtypeStruct((num_tokens * N_EXP, capacity),
                                      jnp.bool_),
        mesh=plsc.VectorSubcoreMesh(core_axis_name="c", subcore_axis_name="s"),
        scratch_types=[
            pltpu.VMEM((ROWS_PER_WORKER,), jnp.int32),
            pltpu.VMEM((CHUNK, capacity), jnp.bool_),
            pltpu.VMEM((rows_per_chunk, capacity), jnp.bool_),
        ],
    )(p2, eye, zeros_chunk)

    cb2 = pl.pallas_call(
        functools.partial(_cb_tc_body, capacity),
        grid=(num_tokens // BTS,),
        in_specs=[pl.BlockSpec((1, num_tokens), lambda i: (0, 0))],
        out_specs=[pl.BlockSpec((BTS, wide), lambda i: (i, 0))],
        out_shape=[jax.ShapeDtypeStruct((num_tokens, wide), jnp.float32)],
        compiler_params=pltpu.CompilerParams(
            dimension_semantics=("parallel",)),
    )(p2)[0]

    used_capacity = uc2[:, 0]
    cb_weight = cb2.reshape(num_tokens, N_EXP, capacity)
    sec_mask = sec2.reshape(num_tokens, N_EXP, capacity)
    return used_capacity, cb_weight, sec_mask


# speedup vs baseline: 3.6972x; 2.1703x over previous
import functools
import math

import jax
import jax.numpy as jnp
from jax.experimental import pallas as pl
from jax.experimental.pallas import tpu as pltpu

N_EXP = 8
TOP_K = 1
CAPACITY_FACTOR = 1.0
MIN_CAPACITY = 4

BT = 256


def _capacity(num_tokens: int) -> int:
    capacity = math.floor(TOP_K * CAPACITY_FACTOR * num_tokens / N_EXP)
    capacity += capacity % 2
    return int(max(capacity, MIN_CAPACITY))


def _router_body(capacity, x_ref, wg_ref, uc_ref, cb_ref, sec_ref,
                 counts_ref):
    i = pl.program_id(0)

    @pl.when(i == 0)
    def _init():
        counts_ref[...] = jnp.zeros_like(counts_ref)

    logits = jax.lax.dot_general(
        x_ref[...], wg_ref[...],
        dimension_numbers=(((1,), (1,)), ((), ())),
        preferred_element_type=jnp.float32,
    )

    m = jnp.max(logits, axis=1, keepdims=True)
    eidx = jax.lax.broadcasted_iota(jnp.int32, (BT, N_EXP), 1)
    e = jnp.min(jnp.where(logits >= m, eidx, N_EXP), axis=1, keepdims=True)
    oh = (eidx == e).astype(jnp.float32)

    ri = jax.lax.broadcasted_iota(jnp.int32, (BT, BT), 0)
    ci = jax.lax.broadcasted_iota(jnp.int32, (BT, BT), 1)
    ltri = (ci < ri).astype(jnp.float32)
    prior = jax.lax.dot_general(
        ltri, oh, dimension_numbers=(((1,), (0,)), ((), ())),
        preferred_element_type=jnp.float32,
    )

    base = counts_ref[...]
    slot_all = prior.astype(jnp.int32) + base
    slots = jnp.sum(jnp.where(eidx == e, slot_all, 0), axis=1, keepdims=True)

    new_counts = base + jnp.sum(oh, axis=0, keepdims=True).astype(jnp.int32)
    counts_ref[...] = new_counts
    uc_ref[...] = jnp.minimum(new_counts, capacity)

    e3 = jax.lax.broadcasted_iota(jnp.int32, (BT, N_EXP, capacity), 1)
    c3 = jax.lax.broadcasted_iota(jnp.int32, (BT, N_EXP, capacity), 2)
    hit = (e3 == e[:, :, None]) & (c3 == slots[:, :, None])
    cb_ref[...] = hit.astype(jnp.float32)
    sec_ref[...] = hit


def kernel(x, w_g):
    num_tokens, n_embd = x.shape
    capacity = _capacity(num_tokens)
    grid = (num_tokens // BT,)
    body = functools.partial(_router_body, capacity)

    uc2, cb_weight, sec_mask = pl.pallas_call(
        body,
        grid=grid,
        in_specs=[
            pl.BlockSpec((BT, n_embd), lambda i: (i, 0)),
            pl.BlockSpec((N_EXP, n_embd), lambda i: (0, 0)),
        ],
        out_specs=[
            pl.BlockSpec((1, N_EXP), lambda i: (0, 0)),
            pl.BlockSpec((BT, N_EXP, capacity), lambda i: (i, 0, 0)),
            pl.BlockSpec((BT, N_EXP, capacity), lambda i: (i, 0, 0)),
        ],
        out_shape=[
            jax.ShapeDtypeStruct((1, N_EXP), jnp.int32),
            jax.ShapeDtypeStruct((num_tokens, N_EXP, capacity), jnp.float32),
            jax.ShapeDtypeStruct((num_tokens, N_EXP, capacity), jnp.bool_),
        ],
        scratch_shapes=[pltpu.VMEM((1, N_EXP), jnp.int32)],
    )(x, w_g)

    return uc2.reshape(N_EXP), cb_weight, sec_mask


# speedup vs baseline: 3.7075x; 1.0028x over previous
import functools
import math

import jax
import jax.numpy as jnp
from jax.experimental import pallas as pl
from jax.experimental.pallas import tpu as pltpu

N_EXP = 8
TOP_K = 1
CAPACITY_FACTOR = 1.0
MIN_CAPACITY = 4

BT = 512


def _capacity(num_tokens: int) -> int:
    capacity = math.floor(TOP_K * CAPACITY_FACTOR * num_tokens / N_EXP)
    capacity += capacity % 2
    return int(max(capacity, MIN_CAPACITY))


def _router_body(capacity, x_ref, wg_ref, uc_ref, cb_ref, sec_ref,
                 counts_ref):
    i = pl.program_id(0)

    @pl.when(i == 0)
    def _init():
        counts_ref[...] = jnp.zeros_like(counts_ref)

    logits = jax.lax.dot_general(
        x_ref[...], wg_ref[...],
        dimension_numbers=(((1,), (1,)), ((), ())),
        preferred_element_type=jnp.float32,
    )

    m = jnp.max(logits, axis=1, keepdims=True)
    eidx = jax.lax.broadcasted_iota(jnp.int32, (BT, N_EXP), 1)
    e = jnp.min(jnp.where(logits >= m, eidx, N_EXP), axis=1, keepdims=True)
    oh = (eidx == e).astype(jnp.float32)

    ri = jax.lax.broadcasted_iota(jnp.int32, (BT, BT), 0)
    ci = jax.lax.broadcasted_iota(jnp.int32, (BT, BT), 1)
    ltri = (ci < ri).astype(jnp.float32)
    prior = jax.lax.dot_general(
        ltri, oh, dimension_numbers=(((1,), (0,)), ((), ())),
        preferred_element_type=jnp.float32,
    )

    base = counts_ref[...]
    slot_all = prior.astype(jnp.int32) + base
    slots = jnp.sum(jnp.where(eidx == e, slot_all, 0), axis=1, keepdims=True)

    new_counts = base + jnp.sum(oh, axis=0, keepdims=True).astype(jnp.int32)
    counts_ref[...] = new_counts
    uc_ref[...] = jnp.minimum(new_counts, capacity)

    e3 = jax.lax.broadcasted_iota(jnp.int32, (BT, N_EXP, capacity), 1)
    c3 = jax.lax.broadcasted_iota(jnp.int32, (BT, N_EXP, capacity), 2)
    hit = (e3 == e[:, :, None]) & (c3 == slots[:, :, None])
    cb_ref[...] = hit.astype(jnp.float32)
    sec_ref[...] = hit


def kernel(x, w_g):
    num_tokens, n_embd = x.shape
    capacity = _capacity(num_tokens)
    grid = (num_tokens // BT,)
    body = functools.partial(_router_body, capacity)

    uc2, cb_weight, sec_mask = pl.pallas_call(
        body,
        grid=grid,
        in_specs=[
            pl.BlockSpec((BT, n_embd), lambda i: (i, 0)),
            pl.BlockSpec((N_EXP, n_embd), lambda i: (0, 0)),
        ],
        out_specs=[
            pl.BlockSpec((1, N_EXP), lambda i: (0, 0)),
            pl.BlockSpec((BT, N_EXP, capacity), lambda i: (i, 0, 0)),
            pl.BlockSpec((BT, N_EXP, capacity), lambda i: (i, 0, 0)),
        ],
        out_shape=[
            jax.ShapeDtypeStruct((1, N_EXP), jnp.int32),
            jax.ShapeDtypeStruct((num_tokens, N_EXP, capacity), jnp.float32),
            jax.ShapeDtypeStruct((num_tokens, N_EXP, capacity), jnp.bool_),
        ],
        scratch_shapes=[pltpu.VMEM((1, N_EXP), jnp.int32)],
    )(x, w_g)

    return uc2.reshape(N_EXP), cb_weight, sec_mask
